# baseline (speedup 1.0000x reference)
import functools

import jax
import jax.numpy as jnp
from jax import lax
from jax.experimental import pallas as pl
from jax.experimental.pallas import tpu as pltpu
from jax.experimental.pallas import tpu_sc as plsc


def _pooled_sum_sc(tokens1d, table2, B, T, D):
    info = plsc.get_sparse_core_info()
    NC, NS, L = info.num_cores, info.num_subcores, info.num_lanes
    NW = NC * NS
    assert B % NW == 0
    b_per_w = B // NW
    n_tok = b_per_w * T
    mesh = plsc.VectorSubcoreMesh(core_axis_name="c", subcore_axis_name="s")
    NBUF = 2

    @functools.partial(
        pl.kernel,
        mesh=mesh,
        compiler_params=pltpu.CompilerParams(
            use_tc_tiling_on_sc=True, needs_layout_passes=False
        ),
        out_type=jax.ShapeDtypeStruct((B // 2, 2 * D), jnp.float32),
        scratch_types=[
            pltpu.VMEM((n_tok,), jnp.int32),
            pltpu.VMEM((n_tok,), jnp.int32),
            pltpu.VMEM((NBUF, T, 2 * D), jnp.float32),
            pltpu.VMEM((b_per_w // 2, 2 * D), jnp.float32),
            pltpu.SemaphoreType.DMA((NBUF,)),
        ],
    )
    def k(tok_hbm, table_hbm, out_hbm, flat_v, idx_v, rows_v, acc_v, sems):
        wid = lax.axis_index("s") * NC + lax.axis_index("c")
        base = wid * n_tok
        pltpu.sync_copy(tok_hbm.at[pl.ds(base, n_tok)], flat_v)

        def build_body(i, _):
            v = flat_v[pl.ds(i * L, L)]
            idx_v[pl.ds(i * L, L)] = lax.shift_right_logical(v, 1)
            return 0

        lax.fori_loop(0, n_tok // L, build_body, 0, unroll=8)

        def gather_start(row, b):
            pltpu.async_copy(
                table_hbm.at[idx_v.at[pl.ds(row * T, T)]],
                rows_v.at[b],
                sems.at[b],
            )

        def gather_wait(row, b):
            pltpu.make_async_copy(
                table_hbm.at[idx_v.at[pl.ds(row * T, T)]],
                rows_v.at[b],
                sems.at[b],
            ).wait()

        for b in range(NBUF):
            gather_start(b, b)

        def group_body(i, _):
            r0 = i * NBUF
            for b in range(NBUF):
                r = r0 + b

                gather_wait(r, b)

                def tok_body(t, acc):
                    p = r * T + t
                    tokv = plsc.load_gather(
                        flat_v, [jnp.full((L,), p, jnp.int32)]
                    )
                    jv = lax.shift_left(
                        lax.bitwise_and(tokv, 1), 6
                    ) + lax.iota(jnp.int32, L)
                    bidx = jnp.full((L,), b, jnp.int32)
                    tidx = jnp.full((L,), t, jnp.int32)
                    return tuple(
                        acc[j]
                        + plsc.load_gather(rows_v, [bidx, tidx, jv + j * L])
                        for j in range(D // L)
                    )

                zeros = tuple(
                    jnp.zeros((L,), jnp.float32) for _ in range(D // L)
                )
                acc = lax.fori_loop(0, T, tok_body, zeros, unroll=4)
                half = lax.mul(lax.bitwise_and(r, 1), D)
                for j in range(D // L):
                    acc_v[
                        lax.shift_right_logical(r, 1),
                        pl.ds(half + j * L, L),
                    ] = acc[j]

                @pl.when(r + NBUF < b_per_w)
                def _():
                    gather_start(r + NBUF, b)

            return 0

        lax.fori_loop(0, b_per_w // NBUF, group_body, 0)
        pltpu.sync_copy(
            acc_v, out_hbm.at[pl.ds(wid * (b_per_w // 2), b_per_w // 2)]
        )

    return k(tokens1d, table2)


def _mlp_tc(pooled_sum, W1, b1, W2, b2, T):
    B, D = pooled_sum.shape

    def body(x_ref, w1_ref, b1_ref, w2_ref, b2_ref, o_ref):
        x = x_ref[...] * (1.0 / T)
        h = jnp.dot(x, w1_ref[...], preferred_element_type=jnp.float32)
        h = jnp.maximum(h + b1_ref[...], 0.0)
        o_ref[...] = (
            jnp.dot(h, w2_ref[...], preferred_element_type=jnp.float32)
            + b2_ref[...]
        )

    out = pl.pallas_call(
        body,
        out_shape=jax.ShapeDtypeStruct((B, 1), jnp.float32),
    )(pooled_sum, W1, b1.reshape(1, D), W2, b2.reshape(1, 1))
    return jnp.squeeze(out, axis=-1)


def kernel(tokens, embed_table, W1, b1, W2, b2):
    B, T = tokens.shape
    V, D = embed_table.shape
    tokens1d = tokens.reshape(B * T)
    table2 = embed_table.reshape(V // 2, 2 * D)
    pooled2 = _pooled_sum_sc(tokens1d, table2, B, T, D)
    pooled_sum = pooled2.reshape(B, D)
    return _mlp_tc(pooled_sum, W1, b1, W2, b2, T)
